# Initial kernel scaffold; baseline (speedup 1.0000x reference)
import jax
import jax.numpy as jnp
from jax.experimental import pallas as pl


def kernel(x, edge_index, W1, b1, W2, b2, Wfc, bfc):
    raise NotImplementedError("write your pallas kernel here")


# speedup vs baseline: 9.0716x; 9.0716x over previous
import jax
import jax.numpy as jnp
from jax import lax
from jax.experimental import pallas as pl
from jax.experimental.pallas import tpu as pltpu
from jax.experimental.pallas import tpu_sc as plsc

N = 10000
E = 320000
D = 128
NPAD = 10240
CHUNK = 128
NW = 32
NCHUNK = NPAD // CHUNK
EPAD = NW * NPAD
NSUB = 16
ROWS_PER_SUB = NPAD // NSUB
DEG_W = 16


def _sc_deg_body(dst_hbm, out_hbm, didx_v, ones_v, zb_v, acc_s):
    cid = lax.axis_index("c")
    sid = lax.axis_index("s")
    wid = sid * 2 + cid

    @pl.loop(0, CHUNK)
    def _(r):
        ones_v[r, :] = jnp.ones((DEG_W,), jnp.float32)
        zb_v[r, :] = jnp.zeros((DEG_W,), jnp.float32)

    row0 = sid * ROWS_PER_SUB
    for k in range(ROWS_PER_SUB // CHUNK):
        pltpu.sync_copy(zb_v, acc_s.at[pl.ds(row0 + k * CHUNK, CHUNK)])
    pltpu.sync_copy(dst_hbm.at[pl.ds(wid * NCHUNK, NCHUNK)], didx_v)
    plsc.subcore_barrier()

    @pl.loop(0, NCHUNK)
    def _(ci):
        pltpu.sync_copy(ones_v, acc_s.at[didx_v.at[ci]], add=True)

    plsc.subcore_barrier()
    pltpu.sync_copy(acc_s.at[pl.ds(row0, ROWS_PER_SUB)],
                    out_hbm.at[cid, pl.ds(row0, ROWS_PER_SUB)])


def _sc_deg(dst2):
    mesh = plsc.VectorSubcoreMesh(core_axis_name="c", subcore_axis_name="s")
    f = pl.kernel(
        _sc_deg_body,
        out_type=jax.ShapeDtypeStruct((2, NPAD, DEG_W), jnp.float32),
        mesh=mesh,
        scratch_types=[
            pltpu.VMEM((NCHUNK, CHUNK), jnp.int32),
            pltpu.VMEM((CHUNK, DEG_W), jnp.float32),
            pltpu.VMEM((CHUNK, DEG_W), jnp.float32),
            pltpu.VMEM_SHARED((NPAD, DEG_W), jnp.float32),
        ],
    )
    return f(dst2)


def _sc_agg_body(g_hbm, src_hbm, dst_hbm, out_hbm, sidx_v, didx_v, rows_v,
                 acc_s, sem):
    cid = lax.axis_index("c")
    sid = lax.axis_index("s")
    wid = sid * 2 + cid

    @pl.loop(0, CHUNK)
    def _(r):
        for c in range(D // 16):
            rows_v[r, pl.ds(c * 16, 16)] = jnp.zeros((16,), jnp.float32)

    row0 = sid * ROWS_PER_SUB
    for k in range(ROWS_PER_SUB // CHUNK):
        pltpu.sync_copy(rows_v, acc_s.at[pl.ds(row0 + k * CHUNK, CHUNK)])
    pltpu.sync_copy(src_hbm.at[pl.ds(wid * NCHUNK, NCHUNK)], sidx_v)
    pltpu.sync_copy(dst_hbm.at[pl.ds(wid * NCHUNK, NCHUNK)], didx_v)
    plsc.subcore_barrier()

    @pl.loop(0, NCHUNK)
    def _(ci):
        pltpu.async_copy(g_hbm.at[sidx_v.at[ci]], rows_v, sem).wait()
        pltpu.sync_copy(rows_v, acc_s.at[didx_v.at[ci]], add=True)

    plsc.subcore_barrier()
    pltpu.sync_copy(acc_s.at[pl.ds(row0, ROWS_PER_SUB)],
                    out_hbm.at[cid, pl.ds(row0, ROWS_PER_SUB)])


def _sc_agg(g, src2, dst2):
    mesh = plsc.VectorSubcoreMesh(core_axis_name="c", subcore_axis_name="s")
    f = pl.kernel(
        _sc_agg_body,
        out_type=jax.ShapeDtypeStruct((2, NPAD, D), jnp.float32),
        mesh=mesh,
        scratch_types=[
            pltpu.VMEM((NCHUNK, CHUNK), jnp.int32),
            pltpu.VMEM((NCHUNK, CHUNK), jnp.int32),
            pltpu.VMEM((CHUNK, D), jnp.float32),
            pltpu.VMEM_SHARED((NPAD, D), jnp.float32),
            pltpu.SemaphoreType.DMA,
        ],
    )
    return f(g, src2, dst2)


def _d_from_degp(degp_ref):
    deg = degp_ref[0, :, 0:1] + degp_ref[1, :, 0:1] + 1.0
    return lax.rsqrt(deg)


def _tc_first_body(x_ref, degp_ref, w_ref, o_ref):
    d = _d_from_degp(degp_ref)
    h = jnp.dot(x_ref[...], w_ref[...],
                precision=lax.Precision.HIGHEST,
                preferred_element_type=jnp.float32)
    o_ref[...] = d * h


def _tc_first(xp, degp, W1):
    return pl.pallas_call(
        _tc_first_body,
        out_shape=jax.ShapeDtypeStruct((NPAD, D), jnp.float32),
    )(xp, degp, W1)


def _tc_mid_body(q_ref, g1_ref, degp_ref, b1_ref, w2_ref, o_ref):
    d = _d_from_degp(degp_ref)
    s = q_ref[0] + q_ref[1] + g1_ref[...]
    z = jnp.maximum(d * s + b1_ref[...], 0.0)
    h = jnp.dot(z, w2_ref[...],
                precision=lax.Precision.HIGHEST,
                preferred_element_type=jnp.float32)
    o_ref[...] = d * h


def _tc_mid(q, g1, degp, b1, W2):
    return pl.pallas_call(
        _tc_mid_body,
        out_shape=jax.ShapeDtypeStruct((NPAD, D), jnp.float32),
    )(q, g1, degp, b1, W2)


def _tc_head_body(r_ref, g2_ref, degp_ref, b2_ref, wfc_ref, bfc_ref, o_ref):
    d = _d_from_degp(degp_ref)
    s = r_ref[0] + r_ref[1] + g2_ref[...]
    z2 = jnp.maximum(d * s + b2_ref[...], 0.0)
    h = jnp.dot(z2, wfc_ref[...],
                precision=lax.Precision.HIGHEST,
                preferred_element_type=jnp.float32)
    o_ref[...] = jax.nn.sigmoid(h + bfc_ref[...])


def _tc_head(r, g2, degp, b2, Wfcp, bfcp):
    return pl.pallas_call(
        _tc_head_body,
        out_shape=jax.ShapeDtypeStruct((NPAD, 8), jnp.float32),
    )(r, g2, degp, b2, Wfcp, bfcp)


def kernel(x, edge_index, W1, b1, W2, b2, Wfc, bfc):
    xp = jnp.concatenate([x, jnp.zeros((NPAD - N, D), x.dtype)], axis=0)
    pad_idx = jnp.full((EPAD - E,), N, jnp.int32)
    src2 = jnp.concatenate([edge_index[0], pad_idx]).reshape(NW * NCHUNK, CHUNK)
    dst2 = jnp.concatenate([edge_index[1], pad_idx]).reshape(NW * NCHUNK, CHUNK)

    degp = _sc_deg(dst2)
    g1 = _tc_first(xp, degp, W1)
    q = _sc_agg(g1, src2, dst2)
    g2 = _tc_mid(q, g1, degp, b1.reshape(1, D), W2)
    r = _sc_agg(g2, src2, dst2)
    Wfcp = jnp.pad(Wfc, ((0, 0), (0, 7)))
    bfcp = jnp.broadcast_to(bfc.reshape(1, 1), (1, 8))
    ph = _tc_head(r, g2, degp, b2.reshape(1, D), Wfcp, bfcp)
    return ph[:N, 0:1]
